# Initial kernel scaffold; baseline (speedup 1.0000x reference)
import functools

import jax
import jax.numpy as jnp
from jax import lax
from jax.experimental import pallas as pl
from jax.experimental.pallas import tpu as pltpu
from jax.experimental.pallas import tpu_sc as plsc

_BUCKETS = 128
_EMB = 32
_NW = 32
_CHUNK = 128
_LANES = 16


def _round_half_even_clip(y):
    k = y.astype(jnp.int32)
    r = y - k.astype(jnp.float32)
    half = jnp.float32(0.5)
    up = (r > half) | ((r == half) & ((k & 1) == 1))
    t = k + jnp.where(up, 1, 0)
    return jnp.minimum(jnp.maximum(t, 0), _BUCKETS - 1)


def _sc_gather(flat_coords, coord_embed):
    n4 = flat_coords.shape[0]
    per_w = n4 // _NW
    n_chunks = per_w // _CHUNK
    groups = _CHUNK // _LANES

    mesh = plsc.VectorSubcoreMesh(core_axis_name="c", subcore_axis_name="s")

    @functools.partial(
        pl.kernel,
        mesh=mesh,
        out_type=jax.ShapeDtypeStruct((n4, _EMB), jnp.float32),
        scratch_types=[
            pltpu.VMEM((per_w,), jnp.float32),
            pltpu.VMEM((n_chunks, _CHUNK), jnp.int32),
            pltpu.VMEM((_CHUNK, _EMB), jnp.float32),
            pltpu.SemaphoreType.DMA,
        ],
    )
    def k(coords_hbm, table_hbm, out_hbm, coords_v, idx_v, rows_v, sem):
        wid = lax.axis_index("s") * 2 + lax.axis_index("c")
        base = wid * per_w
        pltpu.sync_copy(coords_hbm.at[pl.ds(base, per_w)], coords_v)

        def ids_body(j, carry):
            for m in range(groups):
                x = coords_v[pl.ds(j * _CHUNK + m * _LANES, _LANES)]
                idx_v[j, pl.ds(m * _LANES, _LANES)] = _round_half_even_clip(
                    x * jnp.float32(_BUCKETS - 1))
            return carry

        lax.fori_loop(0, n_chunks, ids_body, 0)

        def gather_body(j, carry):
            pltpu.async_copy(table_hbm.at[idx_v.at[j]], rows_v, sem).wait()
            pltpu.sync_copy(rows_v, out_hbm.at[pl.ds(base + j * _CHUNK, _CHUNK)])
            return carry

        lax.fori_loop(0, n_chunks, gather_body, 0)

    return k(flat_coords, coord_embed)


def _tc_mlp(embs, w1, b1, w2, b2, gamma, beta, tile):
    n, d_in = embs.shape
    d_hid = w1.shape[1]
    d_out = w2.shape[1]

    def body(e_ref, w1_ref, b1_ref, w2_ref, b2_ref, g_ref, be_ref, o_ref):
        h = jnp.dot(e_ref[...], w1_ref[...],
                    preferred_element_type=jnp.float32) + b1_ref[...]
        h = jax.nn.gelu(h, approximate=False)
        y = jnp.dot(h, w2_ref[...],
                    preferred_element_type=jnp.float32) + b2_ref[...]
        mu = jnp.mean(y, axis=-1, keepdims=True)
        var = jnp.mean((y - mu) * (y - mu), axis=-1, keepdims=True)
        o_ref[...] = (y - mu) / jnp.sqrt(var + 1e-5) * g_ref[...] + be_ref[...]

    return pl.pallas_call(
        body,
        grid=(n // tile,),
        in_specs=[
            pl.BlockSpec((tile, d_in), lambda i: (i, 0)),
            pl.BlockSpec((d_in, d_hid), lambda i: (0, 0)),
            pl.BlockSpec((1, d_hid), lambda i: (0, 0)),
            pl.BlockSpec((d_hid, d_out), lambda i: (0, 0)),
            pl.BlockSpec((1, d_out), lambda i: (0, 0)),
            pl.BlockSpec((1, d_out), lambda i: (0, 0)),
            pl.BlockSpec((1, d_out), lambda i: (0, 0)),
        ],
        out_specs=pl.BlockSpec((tile, d_out), lambda i: (i, 0)),
        out_shape=jax.ShapeDtypeStruct((n, d_out), jnp.float32),
    )(embs, w1, b1, w2, b2, gamma, beta)


@jax.jit
def kernel(bboxes, coord_embed, W1, b1, W2, b2, gamma, beta):
    b, t, c = bboxes.shape
    embs = _sc_gather(bboxes.reshape(-1), coord_embed)
    embs = embs.reshape(b * t, c * _EMB)
    y = _tc_mlp(embs, W1, b1.reshape(1, -1), W2, b2.reshape(1, -1),
                gamma.reshape(1, -1), beta.reshape(1, -1), tile=512)
    return y.reshape(b, t, W2.shape[1])


# speedup vs baseline: 2.0358x; 2.0358x over previous
import functools

import jax
import jax.numpy as jnp
from jax import lax
from jax.experimental import pallas as pl
from jax.experimental.pallas import tpu as pltpu
from jax.experimental.pallas import tpu_sc as plsc

_BUCKETS = 128
_EMB = 32
_NW = 32
_CHUNK = 128
_LANES = 16


def _round_half_even_clip(y):
    k = y.astype(jnp.int32)
    r = y - k.astype(jnp.float32)
    half = jnp.float32(0.5)
    up = (r > half) | ((r == half) & ((k & 1) == 1))
    t = k + jnp.where(up, 1, 0)
    return jnp.minimum(jnp.maximum(t, 0), _BUCKETS - 1)


def _sc_gather(flat_coords, coord_embed):
    n4 = flat_coords.shape[0]
    per_w = n4 // _NW
    n_chunks = per_w // _CHUNK
    groups = _CHUNK // _LANES

    mesh = plsc.VectorSubcoreMesh(core_axis_name="c", subcore_axis_name="s")

    @functools.partial(
        pl.kernel,
        mesh=mesh,
        out_type=jax.ShapeDtypeStruct((n4, _EMB), jnp.float32),
        scratch_types=[
            pltpu.VMEM((per_w,), jnp.float32),
            pltpu.VMEM((n_chunks, _CHUNK), jnp.int32),
            pltpu.VMEM((_CHUNK, _EMB), jnp.float32),
            pltpu.SemaphoreType.DMA,
        ],
        compiler_params=pltpu.CompilerParams(use_tc_tiling_on_sc=False),
    )
    def k(coords_hbm, table_hbm, out_hbm, coords_v, idx_v, rows_v, sem):
        wid = lax.axis_index("s") * 2 + lax.axis_index("c")
        base = wid * per_w
        pltpu.sync_copy(coords_hbm.at[pl.ds(base, per_w)], coords_v)

        def ids_body(j, carry):
            for m in range(groups):
                x = coords_v[pl.ds(j * _CHUNK + m * _LANES, _LANES)]
                idx_v[j, pl.ds(m * _LANES, _LANES)] = _round_half_even_clip(
                    x * jnp.float32(_BUCKETS - 1))
            return carry

        lax.fori_loop(0, n_chunks, ids_body, 0)

        def gather_body(j, carry):
            pltpu.async_copy(table_hbm.at[idx_v.at[j]], rows_v, sem).wait()
            pltpu.sync_copy(rows_v, out_hbm.at[pl.ds(base + j * _CHUNK, _CHUNK)])
            return carry

        lax.fori_loop(0, n_chunks, gather_body, 0)

    return k(flat_coords, coord_embed)


def _tc_mlp(embs, w1, b1, w2, b2, gamma, beta, tile):
    n, d_in = embs.shape
    d_hid = w1.shape[1]
    d_out = w2.shape[1]

    def body(e_ref, w1_ref, b1_ref, w2_ref, b2_ref, g_ref, be_ref, o_ref):
        h = jnp.dot(e_ref[...], w1_ref[...],
                    preferred_element_type=jnp.float32) + b1_ref[...]
        h = h * 0.5 * (1.0 + lax.erf(h * jnp.float32(0.7071067811865476)))
        y = jnp.dot(h, w2_ref[...],
                    preferred_element_type=jnp.float32) + b2_ref[...]
        mu = jnp.mean(y, axis=-1, keepdims=True)
        var = jnp.mean((y - mu) * (y - mu), axis=-1, keepdims=True)
        o_ref[...] = (y - mu) / jnp.sqrt(var + 1e-5) * g_ref[...] + be_ref[...]

    return pl.pallas_call(
        body,
        grid=(n // tile,),
        in_specs=[
            pl.BlockSpec((tile, d_in), lambda i: (i, 0)),
            pl.BlockSpec((d_in, d_hid), lambda i: (0, 0)),
            pl.BlockSpec((1, d_hid), lambda i: (0, 0)),
            pl.BlockSpec((d_hid, d_out), lambda i: (0, 0)),
            pl.BlockSpec((1, d_out), lambda i: (0, 0)),
            pl.BlockSpec((1, d_out), lambda i: (0, 0)),
            pl.BlockSpec((1, d_out), lambda i: (0, 0)),
        ],
        out_specs=pl.BlockSpec((tile, d_out), lambda i: (i, 0)),
        out_shape=jax.ShapeDtypeStruct((n, d_out), jnp.float32),
    )(embs, w1, b1, w2, b2, gamma, beta)


@jax.jit
def kernel(bboxes, coord_embed, W1, b1, W2, b2, gamma, beta):
    b, t, c = bboxes.shape
    embs = _sc_gather(bboxes.reshape(-1), coord_embed)
    embs = embs.reshape(b * t, c * _EMB)
    y = _tc_mlp(embs, W1, b1.reshape(1, -1), W2, b2.reshape(1, -1),
                gamma.reshape(1, -1), beta.reshape(1, -1), tile=512)
    return y.reshape(b, t, W2.shape[1])
